# baseline (speedup 1.0000x reference)
import dataclasses
import functools

import jax
import jax.numpy as jnp
from jax import lax
from jax.experimental import pallas as pl
from jax.experimental.pallas import tpu as pltpu
from jax.experimental.pallas import tpu_sc as plsc

N_NODES = 10000
N_EDGES = 320000
D = 128

E_BLK = 128
N_BLOCKS = N_EDGES // E_BLK
BLOCKS_PER_CORE = N_BLOCKS // 2
N_SUBCORES = 16
ITERS = (BLOCKS_PER_CORE + N_SUBCORES - 1) // N_SUBCORES
ITERS_PAD = 81
ROWS_MAIN = 624
ROWS_TAIL = N_NODES - N_SUBCORES * ROWS_MAIN


def _mm_body(x_ref, w_ref, src_ref, dst_ref, ew_ref, h_ref, meta_ref):
    h_ref[...] = jnp.dot(x_ref[...], w_ref[...],
                         preferred_element_type=jnp.float32)
    meta_ref[:, 0, :] = src_ref[0]
    meta_ref[:, 1, :] = dst_ref[0]
    meta_ref[:, 2, :] = lax.bitcast_convert_type(ew_ref[0], jnp.int32)


def _matmul_pack(x, W, src, dst, ew):
    grid = 10
    blk = N_NODES // grid
    mblk = N_BLOCKS // grid
    return pl.pallas_call(
        _mm_body,
        grid=(grid,),
        in_specs=[
            pl.BlockSpec((blk, D), lambda i: (i, 0)),
            pl.BlockSpec((D, D), lambda i: (0, 0)),
            pl.BlockSpec((1, mblk, E_BLK), lambda i: (i, 0, 0)),
            pl.BlockSpec((1, mblk, E_BLK), lambda i: (i, 0, 0)),
            pl.BlockSpec((1, mblk, E_BLK), lambda i: (i, 0, 0)),
        ],
        out_specs=[
            pl.BlockSpec((blk, D), lambda i: (i, 0)),
            pl.BlockSpec((mblk, 3, E_BLK), lambda i: (i, 0, 0)),
        ],
        out_shape=[
            jax.ShapeDtypeStruct((N_NODES, D), jnp.float32),
            jax.ShapeDtypeStruct((N_BLOCKS, 3, E_BLK), jnp.int32),
        ],
    )(x, W, src, dst, ew)


def _sc_body(h_hbm, meta_hbm, out_hbm,
             ma, mb_, mc, ra, rb, rc, acc,
             ia, ib, ic, ga, gb, gc, pa, pb, pc):
    c = lax.axis_index("c")
    t = lax.axis_index("s")

    meta = (ma, mb_, mc)
    rows = (ra, rb, rc)
    isem = (ia, ib, ic)
    gsem = (ga, gb, gc)
    ssem = (pa, pb, pc)

    def idx_start(rel, b):
        base = c * BLOCKS_PER_CORE + rel
        pltpu.async_copy(meta_hbm.at[base], meta[b], isem[b])

    def idx_wait(rel, b):
        base = c * BLOCKS_PER_CORE + rel
        pltpu.make_async_copy(meta_hbm.at[base], meta[b], isem[b]).wait()

    idx_start(t, 0)
    idx_start(t + N_SUBCORES, 1)

    @pl.loop(0, E_BLK)
    def _zero_rows(r):
        for j in range(D // 16):
            ra[r, pl.ds(16 * j, 16)] = jnp.zeros((16,), jnp.float32)

    for k, sz in ((0, 128), (128, 128), (256, 128), (384, 128), (512, 112)):
        pltpu.sync_copy(ra.at[pl.ds(0, sz)],
                        acc.at[pl.ds(t * ROWS_MAIN + k, sz)])

    @pl.when(t == N_SUBCORES - 1)
    def _zero_tail():
        pltpu.sync_copy(ra.at[pl.ds(0, ROWS_TAIL)],
                        acc.at[pl.ds(N_SUBCORES * ROWS_MAIN, ROWS_TAIL)])

    def gather_start(b):
        pltpu.async_copy(h_hbm.at[meta[b].at[0]], rows[b], gsem[b])

    def gather_wait(b):
        pltpu.make_async_copy(h_hbm.at[meta[b].at[0]], rows[b],
                              gsem[b]).wait()

    def scale(b):
        buf = rows[b]
        wref = meta[b]

        @pl.loop(0, E_BLK, step=4)
        def _scale(e):
            for v in range(4):
                w16i = plsc.load_gather(
                    wref, [jnp.full((16,), 2, jnp.int32),
                           jnp.full((16,), e + v, jnp.int32)])
                w16 = plsc.bitcast(w16i, jnp.float32)
                for u in range(D // 16):
                    sl = pl.ds(16 * u, 16)
                    buf[e + v, sl] = buf[e + v, sl] * w16

    def scatter_start(b):
        pltpu.async_copy(rows[b], acc.at[meta[b].at[1]], ssem[b], add=True)

    def scatter_wait(b):
        pltpu.make_async_copy(rows[b], acc.at[meta[b].at[1]],
                              ssem[b]).wait()

    idx_wait(t, 0)
    gather_start(0)
    plsc.subcore_barrier()

    @pl.loop(0, ITERS_PAD, step=3)
    def _edge_iter(i):
        for u in range(3):
            k = i + u
            S = u
            P = (u + 1) % 3
            Q = (u + 2) % 3
            rel = k * N_SUBCORES + t

            @pl.when(rel + N_SUBCORES < BLOCKS_PER_CORE)
            def _launch_next():
                idx_wait(rel + N_SUBCORES, P)
                gather_start(P)

            @pl.when(rel < BLOCKS_PER_CORE)
            def _process():
                gather_wait(S)
                scale(S)
                scatter_start(S)

            @pl.when((k >= 1) & (rel - N_SUBCORES < BLOCKS_PER_CORE))
            def _retire_prev():
                scatter_wait(Q)

            @pl.when(rel + 2 * N_SUBCORES < BLOCKS_PER_CORE)
            def _prefetch_idx():
                idx_start(rel + 2 * N_SUBCORES, Q)

    plsc.subcore_barrier()

    d0 = t * ROWS_MAIN
    pltpu.sync_copy(acc.at[pl.ds(d0, ROWS_MAIN)],
                    out_hbm.at[c, pl.ds(d0, ROWS_MAIN)])

    @pl.when(t == N_SUBCORES - 1)
    def _drain_tail():
        d1 = N_SUBCORES * ROWS_MAIN
        pltpu.sync_copy(acc.at[pl.ds(d1, ROWS_TAIL)],
                        out_hbm.at[c, pl.ds(d1, ROWS_TAIL)])


def _sc_aggregate(h, meta):
    mesh = plsc.VectorSubcoreMesh(core_axis_name="c", subcore_axis_name="s")
    cp = pltpu.CompilerParams()
    if "needs_layout_passes" in pltpu.CompilerParams.__dataclass_fields__:
        cp = dataclasses.replace(cp, needs_layout_passes=False)
    kern = pl.kernel(
        _sc_body,
        out_type=jax.ShapeDtypeStruct((2, N_NODES, D), jnp.float32),
        mesh=mesh,
        scratch_types=[
            pltpu.VMEM((3, E_BLK), jnp.int32),
            pltpu.VMEM((3, E_BLK), jnp.int32),
            pltpu.VMEM((3, E_BLK), jnp.int32),
            pltpu.VMEM((E_BLK, D), jnp.float32),
            pltpu.VMEM((E_BLK, D), jnp.float32),
            pltpu.VMEM((E_BLK, D), jnp.float32),
            pltpu.VMEM_SHARED((N_NODES, D), jnp.float32),
            pltpu.SemaphoreType.DMA,
            pltpu.SemaphoreType.DMA,
            pltpu.SemaphoreType.DMA,
            pltpu.SemaphoreType.DMA,
            pltpu.SemaphoreType.DMA,
            pltpu.SemaphoreType.DMA,
            pltpu.SemaphoreType.DMA,
            pltpu.SemaphoreType.DMA,
            pltpu.SemaphoreType.DMA,
        ],
        compiler_params=cp,
    )
    return kern(h, meta)


def _add_body(p_ref, o_ref):
    o_ref[...] = p_ref[0] + p_ref[1]


def _sum_partials(partials):
    grid = 10
    blk = N_NODES // grid
    return pl.pallas_call(
        _add_body,
        grid=(grid,),
        in_specs=[pl.BlockSpec((2, blk, D), lambda i: (0, i, 0))],
        out_specs=pl.BlockSpec((blk, D), lambda i: (i, 0)),
        out_shape=jax.ShapeDtypeStruct((N_NODES, D), jnp.float32),
    )(partials)


def kernel(x, edge_index, edge_weight, W):
    grid = 10
    mblk = N_BLOCKS // grid
    src = edge_index[0].reshape(grid, mblk, E_BLK)
    dst = edge_index[1].reshape(grid, mblk, E_BLK)
    ew = edge_weight.reshape(grid, mblk, E_BLK)
    h, meta = _matmul_pack(x, W, src, dst, ew)
    partials = _sc_aggregate(h, meta)
    return _sum_partials(partials)


# speedup vs baseline: 1.0399x; 1.0399x over previous
import dataclasses
import functools

import jax
import jax.numpy as jnp
from jax import lax
from jax.experimental import pallas as pl
from jax.experimental.pallas import tpu as pltpu
from jax.experimental.pallas import tpu_sc as plsc

N_NODES = 10000
N_EDGES = 320000
D = 128

E_BLK = 128
N_BLOCKS = N_EDGES // E_BLK
BLOCKS_PER_CORE = N_BLOCKS // 2
N_SUBCORES = 16
ITERS = (BLOCKS_PER_CORE + N_SUBCORES - 1) // N_SUBCORES
ITERS_PAD = 81
ROWS_MAIN = 624
ROWS_TAIL = N_NODES - N_SUBCORES * ROWS_MAIN


def _mm_body(x_ref, w_ref, h_ref):
    h_ref[...] = jnp.dot(x_ref[...], w_ref[...],
                         preferred_element_type=jnp.float32)


def _matmul(x, W):
    grid = 10
    blk = N_NODES // grid
    return pl.pallas_call(
        _mm_body,
        grid=(grid,),
        in_specs=[
            pl.BlockSpec((blk, D), lambda i: (i, 0)),
            pl.BlockSpec((D, D), lambda i: (0, 0)),
        ],
        out_specs=pl.BlockSpec((blk, D), lambda i: (i, 0)),
        out_shape=jax.ShapeDtypeStruct((N_NODES, D), jnp.float32),
    )(x, W)


def _sc_body(h_hbm, meta_hbm, out_hbm,
             ma, mb_, mc, ra, rb, rc, acc,
             ia, ib, ic, ga, gb, gc, pa, pb, pc):
    c = lax.axis_index("c")
    t = lax.axis_index("s")

    meta = (ma, mb_, mc)
    rows = (ra, rb, rc)
    isem = (ia, ib, ic)
    gsem = (ga, gb, gc)
    ssem = (pa, pb, pc)

    def idx_start(rel, b):
        base = c * BLOCKS_PER_CORE + rel
        pltpu.async_copy(meta_hbm.at[base], meta[b], isem[b])

    def idx_wait(rel, b):
        base = c * BLOCKS_PER_CORE + rel
        pltpu.make_async_copy(meta_hbm.at[base], meta[b], isem[b]).wait()

    idx_start(t, 0)
    idx_start(t + N_SUBCORES, 1)

    @pl.loop(0, E_BLK)
    def _zero_rows(r):
        for j in range(D // 16):
            ra[r, pl.ds(16 * j, 16)] = jnp.zeros((16,), jnp.float32)

    for k, sz in ((0, 128), (128, 128), (256, 128), (384, 128), (512, 112)):
        pltpu.sync_copy(ra.at[pl.ds(0, sz)],
                        acc.at[pl.ds(t * ROWS_MAIN + k, sz)])

    @pl.when(t == N_SUBCORES - 1)
    def _zero_tail():
        pltpu.sync_copy(ra.at[pl.ds(0, ROWS_TAIL)],
                        acc.at[pl.ds(N_SUBCORES * ROWS_MAIN, ROWS_TAIL)])

    def gather_start(b):
        pltpu.async_copy(h_hbm.at[meta[b].at[0]], rows[b], gsem[b])

    def gather_wait(b):
        pltpu.make_async_copy(h_hbm.at[meta[b].at[0]], rows[b],
                              gsem[b]).wait()

    def scale(b):
        buf = rows[b]
        wref = meta[b]

        @pl.loop(0, E_BLK, step=4)
        def _scale(e):
            for v in range(4):
                w16i = plsc.load_gather(
                    wref, [jnp.full((16,), 2, jnp.int32),
                           jnp.full((16,), e + v, jnp.int32)])
                w16 = plsc.bitcast(w16i, jnp.float32)
                for u in range(D // 16):
                    sl = pl.ds(16 * u, 16)
                    buf[e + v, sl] = buf[e + v, sl] * w16

    def scatter_start(b):
        pltpu.async_copy(rows[b], acc.at[meta[b].at[1]], ssem[b], add=True)

    def scatter_wait(b):
        pltpu.make_async_copy(rows[b], acc.at[meta[b].at[1]],
                              ssem[b]).wait()

    idx_wait(t, 0)
    gather_start(0)
    plsc.subcore_barrier()

    @pl.loop(0, ITERS_PAD, step=3)
    def _edge_iter(i):
        for u in range(3):
            k = i + u
            S = u
            P = (u + 1) % 3
            Q = (u + 2) % 3
            rel = k * N_SUBCORES + t

            @pl.when(rel + N_SUBCORES < BLOCKS_PER_CORE)
            def _launch_next():
                idx_wait(rel + N_SUBCORES, P)
                gather_start(P)

            @pl.when(rel < BLOCKS_PER_CORE)
            def _process():
                gather_wait(S)
                scale(S)
                scatter_start(S)

            @pl.when((k >= 1) & (rel - N_SUBCORES < BLOCKS_PER_CORE))
            def _retire_prev():
                scatter_wait(Q)

            @pl.when(rel + 2 * N_SUBCORES < BLOCKS_PER_CORE)
            def _prefetch_idx():
                idx_start(rel + 2 * N_SUBCORES, Q)

    plsc.subcore_barrier()

    d0 = t * ROWS_MAIN
    pltpu.sync_copy(acc.at[pl.ds(d0, ROWS_MAIN)],
                    out_hbm.at[c, pl.ds(d0, ROWS_MAIN)])

    @pl.when(t == N_SUBCORES - 1)
    def _drain_tail():
        d1 = N_SUBCORES * ROWS_MAIN
        pltpu.sync_copy(acc.at[pl.ds(d1, ROWS_TAIL)],
                        out_hbm.at[c, pl.ds(d1, ROWS_TAIL)])


def _sc_aggregate(h, meta):
    mesh = plsc.VectorSubcoreMesh(core_axis_name="c", subcore_axis_name="s")
    cp = pltpu.CompilerParams()
    if "needs_layout_passes" in pltpu.CompilerParams.__dataclass_fields__:
        cp = dataclasses.replace(cp, needs_layout_passes=False)
    kern = pl.kernel(
        _sc_body,
        out_type=jax.ShapeDtypeStruct((2, N_NODES, D), jnp.float32),
        mesh=mesh,
        scratch_types=[
            pltpu.VMEM((3, E_BLK), jnp.int32),
            pltpu.VMEM((3, E_BLK), jnp.int32),
            pltpu.VMEM((3, E_BLK), jnp.int32),
            pltpu.VMEM((E_BLK, D), jnp.float32),
            pltpu.VMEM((E_BLK, D), jnp.float32),
            pltpu.VMEM((E_BLK, D), jnp.float32),
            pltpu.VMEM_SHARED((N_NODES, D), jnp.float32),
            pltpu.SemaphoreType.DMA,
            pltpu.SemaphoreType.DMA,
            pltpu.SemaphoreType.DMA,
            pltpu.SemaphoreType.DMA,
            pltpu.SemaphoreType.DMA,
            pltpu.SemaphoreType.DMA,
            pltpu.SemaphoreType.DMA,
            pltpu.SemaphoreType.DMA,
            pltpu.SemaphoreType.DMA,
        ],
        compiler_params=cp,
    )
    return kern(h, meta)


def _add_body(p_ref, o_ref):
    o_ref[...] = p_ref[0] + p_ref[1]


def _sum_partials(partials):
    grid = 10
    blk = N_NODES // grid
    return pl.pallas_call(
        _add_body,
        grid=(grid,),
        in_specs=[pl.BlockSpec((2, blk, D), lambda i: (0, i, 0))],
        out_specs=pl.BlockSpec((blk, D), lambda i: (i, 0)),
        out_shape=jax.ShapeDtypeStruct((N_NODES, D), jnp.float32),
    )(partials)


def kernel(x, edge_index, edge_weight, W):
    h = _matmul(x, W)
    srcr = edge_index[0].reshape(N_BLOCKS, 1, E_BLK)
    dstr = edge_index[1].reshape(N_BLOCKS, 1, E_BLK)
    wbits = lax.bitcast_convert_type(edge_weight, jnp.int32)
    wr = wbits.reshape(N_BLOCKS, 1, E_BLK)
    meta = jnp.concatenate([srcr, dstr, wr], axis=1)
    partials = _sc_aggregate(h, meta)
    return _sum_partials(partials)
